# Initial kernel scaffold; baseline (speedup 1.0000x reference)
import jax
import jax.numpy as jnp
from jax.experimental import pallas as pl


def kernel(vectors, codebook):
    raise NotImplementedError("write your pallas kernel here")


# speedup vs baseline: 1.5153x; 1.5153x over previous
import functools

import jax
import jax.numpy as jnp
from jax.experimental import pallas as pl

K = 8
B = 1024
SUBD = 64
BV = 1024


def _pq_kernel(x_ref, cbt_ref, out_ref):
    cols = []
    for k in range(K):
        xk = x_ref[:, k * SUBD:(k + 1) * SUBD]
        ck = cbt_ref[k]
        x2 = jnp.sum(xk * xk, axis=1, keepdims=True)
        c2 = jnp.sum(ck * ck, axis=0, keepdims=True)
        cross = jax.lax.dot(
            xk, ck,
            precision=jax.lax.Precision.DEFAULT,
            preferred_element_type=jnp.float32,
        )
        d2 = (x2 - 2.0 * cross) + c2
        d2 = jnp.maximum(d2, 0.0)
        m = jnp.min(d2, axis=1, keepdims=True)
        lane = jax.lax.broadcasted_iota(jnp.int32, d2.shape, 1)
        idx = jnp.min(jnp.where(d2 <= m, lane, B), axis=1, keepdims=True)
        cols.append(idx)
    out_ref[:, :] = jnp.concatenate(cols, axis=1)


@jax.jit
def kernel(vectors, codebook):
    v, _ = vectors.shape
    cbt = jnp.swapaxes(codebook, 1, 2)
    out = pl.pallas_call(
        _pq_kernel,
        grid=(v // BV,),
        in_specs=[
            pl.BlockSpec((BV, K * SUBD), lambda i: (i, 0)),
            pl.BlockSpec((K, SUBD, B), lambda i: (0, 0, 0)),
        ],
        out_specs=pl.BlockSpec((BV, K), lambda i: (i, 0)),
        out_shape=jax.ShapeDtypeStruct((v, K), jnp.int32),
    )(vectors, cbt)
    return out.astype(jnp.uint8)


# speedup vs baseline: 1.8396x; 1.2140x over previous
import jax
import jax.numpy as jnp
from jax.experimental import pallas as pl
from jax.experimental.pallas import tpu as pltpu

K = 8
B = 1024
SUBD = 64
BV = 1024


def _prep_kernel(cbt_ref, cbs_ref, c2_ref):
    cbt = cbt_ref[...]
    cbs_ref[...] = cbt * -2.0
    c2_ref[...] = jnp.sum(cbt * cbt, axis=1)


def _pq_kernel(x_ref, cbs_ref, c2_ref, out_ref):
    cols = []
    for k in range(K):
        xk = x_ref[:, k * SUBD:(k + 1) * SUBD]
        x2 = jnp.sum(xk * xk, axis=1, keepdims=True)
        ncross = jax.lax.dot(
            xk, cbs_ref[k],
            precision=jax.lax.Precision.DEFAULT,
            preferred_element_type=jnp.float32,
        )
        d2 = (x2 + ncross) + c2_ref[k:k + 1, :]
        m = jnp.min(d2, axis=1, keepdims=True)
        lane = jax.lax.broadcasted_iota(jnp.int32, d2.shape, 1)
        idx = jnp.min(jnp.where(d2 <= m, lane, B), axis=1, keepdims=True)
        cols.append(idx)
    out_ref[:, :] = jnp.concatenate(cols, axis=1)


@jax.jit
def kernel(vectors, codebook):
    v, _ = vectors.shape
    cbt = jnp.swapaxes(codebook, 1, 2)
    cbs, c2 = pl.pallas_call(
        _prep_kernel,
        out_shape=[
            jax.ShapeDtypeStruct((K, SUBD, B), jnp.float32),
            jax.ShapeDtypeStruct((K, B), jnp.float32),
        ],
    )(cbt)
    out = pl.pallas_call(
        _pq_kernel,
        grid=(v // BV,),
        in_specs=[
            pl.BlockSpec((BV, K * SUBD), lambda i: (i, 0)),
            pl.BlockSpec((K, SUBD, B), lambda i: (0, 0, 0)),
            pl.BlockSpec((K, B), lambda i: (0, 0)),
        ],
        out_specs=pl.BlockSpec((BV, K), lambda i: (i, 0)),
        out_shape=jax.ShapeDtypeStruct((v, K), jnp.int32),
        compiler_params=pltpu.CompilerParams(
            dimension_semantics=("parallel",),
        ),
    )(vectors, cbs, c2)
    return out.astype(jnp.uint8)


# speedup vs baseline: 2.2768x; 1.2377x over previous
import jax
import jax.numpy as jnp
from jax.experimental import pallas as pl
from jax.experimental.pallas import tpu as pltpu

K = 8
B = 1024
SUBD = 64
BV = 1024


def _prep_kernel(cbt_ref, cbs_ref, c2_ref):
    cbt = cbt_ref[...]
    cbs_ref[...] = cbt * -2.0
    c2_ref[...] = jnp.sum(cbt * cbt, axis=1)


def _pq_kernel(x_ref, cbs_ref, c2_ref, out_ref):
    cols = []
    for k in range(K):
        xk = x_ref[:, k * SUBD:(k + 1) * SUBD]
        ncross = jax.lax.dot(
            xk, cbs_ref[k],
            precision=jax.lax.Precision.DEFAULT,
            preferred_element_type=jnp.float32,
        )
        d2 = ncross + c2_ref[k:k + 1, :]
        m = jnp.min(d2, axis=1, keepdims=True)
        lane = jax.lax.broadcasted_iota(jnp.int32, d2.shape, 1)
        idx = jnp.min(jnp.where(d2 <= m, lane, B), axis=1, keepdims=True)
        cols.append(idx)
    out_ref[:, :] = jnp.concatenate(cols, axis=1)


@jax.jit
def kernel(vectors, codebook):
    v, _ = vectors.shape
    cbt = jnp.swapaxes(codebook, 1, 2)
    cbs, c2 = pl.pallas_call(
        _prep_kernel,
        out_shape=[
            jax.ShapeDtypeStruct((K, SUBD, B), jnp.float32),
            jax.ShapeDtypeStruct((K, B), jnp.float32),
        ],
    )(cbt)
    out = pl.pallas_call(
        _pq_kernel,
        grid=(v // BV,),
        in_specs=[
            pl.BlockSpec((BV, K * SUBD), lambda i: (i, 0)),
            pl.BlockSpec((K, SUBD, B), lambda i: (0, 0, 0)),
            pl.BlockSpec((K, B), lambda i: (0, 0)),
        ],
        out_specs=pl.BlockSpec((BV, K), lambda i: (i, 0)),
        out_shape=jax.ShapeDtypeStruct((v, K), jnp.int32),
        compiler_params=pltpu.CompilerParams(
            dimension_semantics=("parallel",),
        ),
    )(vectors, cbs, c2)
    return out.astype(jnp.uint8)


# speedup vs baseline: 2.9104x; 1.2783x over previous
import jax
import jax.numpy as jnp
from jax.experimental import pallas as pl
from jax.experimental.pallas import tpu as pltpu

K = 8
B = 1024
SUBD = 64
BV = 1024


def _prep_kernel(cbt_ref, cbs_ref, c2_ref):
    cbt = cbt_ref[...]
    cbs_ref[...] = cbt * -2.0
    c2_ref[...] = jnp.sum(cbt * cbt, axis=1)


def _pq_kernel(x_ref, cbs_ref, c2_ref, out_ref):
    cols = []
    for k in range(K):
        xk = x_ref[:, k * SUBD:(k + 1) * SUBD]
        ncross = jax.lax.dot(
            xk, cbs_ref[k],
            precision=jax.lax.Precision.DEFAULT,
            preferred_element_type=jnp.float32,
        )
        d2 = ncross + c2_ref[k:k + 1, :]
        idx = jnp.argmin(d2, axis=1, keepdims=True).astype(jnp.int32)
        cols.append(idx)
    out_ref[:, :] = jnp.concatenate(cols, axis=1)


@jax.jit
def kernel(vectors, codebook):
    v, _ = vectors.shape
    cbt = jnp.swapaxes(codebook, 1, 2)
    cbs, c2 = pl.pallas_call(
        _prep_kernel,
        out_shape=[
            jax.ShapeDtypeStruct((K, SUBD, B), jnp.float32),
            jax.ShapeDtypeStruct((K, B), jnp.float32),
        ],
    )(cbt)
    out = pl.pallas_call(
        _pq_kernel,
        grid=(v // BV,),
        in_specs=[
            pl.BlockSpec((BV, K * SUBD), lambda i: (i, 0)),
            pl.BlockSpec((K, SUBD, B), lambda i: (0, 0, 0)),
            pl.BlockSpec((K, B), lambda i: (0, 0)),
        ],
        out_specs=pl.BlockSpec((BV, K), lambda i: (i, 0)),
        out_shape=jax.ShapeDtypeStruct((v, K), jnp.int32),
        compiler_params=pltpu.CompilerParams(
            dimension_semantics=("parallel",),
        ),
    )(vectors, cbs, c2)
    return out.astype(jnp.uint8)


# speedup vs baseline: 3.2652x; 1.1219x over previous
import jax
import jax.numpy as jnp
from jax.experimental import pallas as pl
from jax.experimental.pallas import tpu as pltpu

K = 8
B = 1024
SUBD = 64
AUG = SUBD + 8
BV = 1024


def _prep_kernel(cbt_ref, cbs_ref):
    cbt = cbt_ref[...]
    c2 = jnp.sum(cbt * cbt, axis=1)
    hi = c2.astype(jnp.bfloat16).astype(jnp.float32)
    r1 = c2 - hi
    mid = r1.astype(jnp.bfloat16).astype(jnp.float32)
    lo = (r1 - mid).astype(jnp.bfloat16).astype(jnp.float32)
    cbs_ref[:, :SUBD, :] = cbt * -2.0
    cbs_ref[:, SUBD:SUBD + 3, :] = jnp.stack([hi, mid, lo], axis=1)
    cbs_ref[:, SUBD + 3:, :] = jnp.zeros((K, 5, B), jnp.float32)


def _pq_kernel(x_ref, cbs_ref, out_ref):
    one = jnp.ones((BV, 1), jnp.float32)
    zero = jnp.zeros((BV, 1), jnp.float32)
    aug = jnp.concatenate([one, one, one, zero, zero, zero, zero, zero],
                          axis=1)
    cols = []
    for k in range(K):
        xa = jnp.concatenate(
            [x_ref[:, k * SUBD:(k + 1) * SUBD], aug], axis=1)
        d2 = jax.lax.dot(
            xa, cbs_ref[k],
            precision=jax.lax.Precision.DEFAULT,
            preferred_element_type=jnp.float32,
        )
        idx = jnp.argmin(d2, axis=1, keepdims=True).astype(jnp.int32)
        cols.append(idx)
    out_ref[:, :] = jnp.concatenate(cols, axis=1)


@jax.jit
def kernel(vectors, codebook):
    v, _ = vectors.shape
    cbt = jnp.swapaxes(codebook, 1, 2)
    cbs = pl.pallas_call(
        _prep_kernel,
        out_shape=jax.ShapeDtypeStruct((K, AUG, B), jnp.float32),
    )(cbt)
    out = pl.pallas_call(
        _pq_kernel,
        grid=(v // BV,),
        in_specs=[
            pl.BlockSpec((BV, K * SUBD), lambda i: (i, 0)),
            pl.BlockSpec((K, AUG, B), lambda i: (0, 0, 0)),
        ],
        out_specs=pl.BlockSpec((BV, K), lambda i: (i, 0)),
        out_shape=jax.ShapeDtypeStruct((v, K), jnp.int32),
        compiler_params=pltpu.CompilerParams(
            dimension_semantics=("parallel",),
        ),
    )(vectors, cbs)
    return out.astype(jnp.uint8)


# speedup vs baseline: 3.4606x; 1.0599x over previous
import jax
import jax.numpy as jnp
from jax.experimental import pallas as pl
from jax.experimental.pallas import tpu as pltpu

K = 8
B = 1024
SUBD = 64
AUG = SUBD + 8
BV = 2048


def _prep_kernel(cbt_ref, cbs_ref):
    cbt = cbt_ref[...]
    c2 = jnp.sum(cbt * cbt, axis=1)
    hi = c2.astype(jnp.bfloat16).astype(jnp.float32)
    r1 = c2 - hi
    mid = r1.astype(jnp.bfloat16).astype(jnp.float32)
    lo = (r1 - mid).astype(jnp.bfloat16).astype(jnp.float32)
    cbs_ref[:, :SUBD, :] = cbt * -2.0
    cbs_ref[:, SUBD:SUBD + 3, :] = jnp.stack([hi, mid, lo], axis=1)
    cbs_ref[:, SUBD + 3:, :] = jnp.zeros((K, 5, B), jnp.float32)


def _pq_kernel(x_ref, cbs_ref, out_ref):
    one = jnp.ones((BV, 1), jnp.float32)
    zero = jnp.zeros((BV, 1), jnp.float32)
    aug = jnp.concatenate([one, one, one, zero, zero, zero, zero, zero],
                          axis=1)
    cols = []
    for k in range(K):
        xa = jnp.concatenate(
            [x_ref[:, k * SUBD:(k + 1) * SUBD], aug], axis=1)
        d2 = jax.lax.dot(
            xa, cbs_ref[k],
            precision=jax.lax.Precision.DEFAULT,
            preferred_element_type=jnp.float32,
        )
        idx = jnp.argmin(d2, axis=1, keepdims=True).astype(jnp.int32)
        cols.append(idx)
    out_ref[:, :] = jnp.concatenate(cols, axis=1)


@jax.jit
def kernel(vectors, codebook):
    v, _ = vectors.shape
    cbt = jnp.swapaxes(codebook, 1, 2)
    cbs = pl.pallas_call(
        _prep_kernel,
        out_shape=jax.ShapeDtypeStruct((K, AUG, B), jnp.float32),
    )(cbt)
    out = pl.pallas_call(
        _pq_kernel,
        grid=(v // BV,),
        in_specs=[
            pl.BlockSpec((BV, K * SUBD), lambda i: (i, 0)),
            pl.BlockSpec((K, AUG, B), lambda i: (0, 0, 0)),
        ],
        out_specs=pl.BlockSpec((BV, K), lambda i: (i, 0)),
        out_shape=jax.ShapeDtypeStruct((v, K), jnp.int32),
        compiler_params=pltpu.CompilerParams(
            dimension_semantics=("parallel",),
        ),
    )(vectors, cbs)
    return out.astype(jnp.uint8)


# speedup vs baseline: 3.6574x; 1.0568x over previous
import jax
import jax.numpy as jnp
from jax.experimental import pallas as pl
from jax.experimental.pallas import tpu as pltpu

K = 8
B = 1024
SUBD = 64
AUG = SUBD + 8
BV = 4096


def _prep_kernel(cbt_ref, cbs_ref):
    cbt = cbt_ref[...]
    c2 = jnp.sum(cbt * cbt, axis=1)
    hi = c2.astype(jnp.bfloat16).astype(jnp.float32)
    r1 = c2 - hi
    mid = r1.astype(jnp.bfloat16).astype(jnp.float32)
    lo = (r1 - mid).astype(jnp.bfloat16).astype(jnp.float32)
    cbs_ref[:, :SUBD, :] = cbt * -2.0
    cbs_ref[:, SUBD:SUBD + 3, :] = jnp.stack([hi, mid, lo], axis=1)
    cbs_ref[:, SUBD + 3:, :] = jnp.zeros((K, 5, B), jnp.float32)


def _pq_kernel(x_ref, cbs_ref, out_ref):
    one = jnp.ones((BV, 1), jnp.float32)
    zero = jnp.zeros((BV, 1), jnp.float32)
    aug = jnp.concatenate([one, one, one, zero, zero, zero, zero, zero],
                          axis=1)
    cols = []
    for k in range(K):
        xa = jnp.concatenate(
            [x_ref[:, k * SUBD:(k + 1) * SUBD], aug], axis=1)
        d2 = jax.lax.dot(
            xa, cbs_ref[k],
            precision=jax.lax.Precision.DEFAULT,
            preferred_element_type=jnp.float32,
        )
        idx = jnp.argmin(d2, axis=1, keepdims=True).astype(jnp.int32)
        cols.append(idx)
    out_ref[:, :] = jnp.concatenate(cols, axis=1)


@jax.jit
def kernel(vectors, codebook):
    v, _ = vectors.shape
    cbt = jnp.swapaxes(codebook, 1, 2)
    cbs = pl.pallas_call(
        _prep_kernel,
        out_shape=jax.ShapeDtypeStruct((K, AUG, B), jnp.float32),
    )(cbt)
    out = pl.pallas_call(
        _pq_kernel,
        grid=(v // BV,),
        in_specs=[
            pl.BlockSpec((BV, K * SUBD), lambda i: (i, 0)),
            pl.BlockSpec((K, AUG, B), lambda i: (0, 0, 0)),
        ],
        out_specs=pl.BlockSpec((BV, K), lambda i: (i, 0)),
        out_shape=jax.ShapeDtypeStruct((v, K), jnp.int32),
        compiler_params=pltpu.CompilerParams(
            dimension_semantics=("parallel",),
        ),
    )(vectors, cbs)
    return out.astype(jnp.uint8)


# speedup vs baseline: 3.7994x; 1.0388x over previous
import jax
import jax.numpy as jnp
from jax.experimental import pallas as pl
from jax.experimental.pallas import tpu as pltpu

K = 8
B = 1024
SUBD = 64
AUG = SUBD + 8
BV = 4096


def _prep_kernel(cbt_ref, cbs_ref):
    cbt = cbt_ref[...]
    c2 = jnp.sum(cbt * cbt, axis=1)
    hi = c2.astype(jnp.bfloat16).astype(jnp.float32)
    r1 = c2 - hi
    mid = r1.astype(jnp.bfloat16).astype(jnp.float32)
    lo = (r1 - mid).astype(jnp.bfloat16).astype(jnp.float32)
    cbs_ref[:, :SUBD, :] = cbt * -2.0
    cbs_ref[:, SUBD:SUBD + 3, :] = jnp.stack([hi, mid, lo], axis=1)
    cbs_ref[:, SUBD + 3:, :] = jnp.zeros((K, 5, B), jnp.float32)


def _pq_kernel(x_ref, cbs_ref, out_ref):
    one = jnp.ones((BV, 1), jnp.float32)
    zero = jnp.zeros((BV, 1), jnp.float32)
    aug = jnp.concatenate([one, one, one, zero, zero, zero, zero, zero],
                          axis=1)
    cols = []
    for k in range(K):
        xa = jnp.concatenate(
            [x_ref[:, k * SUBD:(k + 1) * SUBD], aug], axis=1)
        d2 = jax.lax.dot(
            xa, cbs_ref[k],
            precision=jax.lax.Precision.DEFAULT,
            preferred_element_type=jnp.float32,
        )
        idx = jnp.argmin(d2, axis=1, keepdims=True).astype(jnp.int32)
        cols.append(idx)
    out_ref[:, :] = jnp.concatenate(cols, axis=1).astype(jnp.uint8)


@jax.jit
def kernel(vectors, codebook):
    v, _ = vectors.shape
    cbt = jnp.swapaxes(codebook, 1, 2)
    cbs = pl.pallas_call(
        _prep_kernel,
        out_shape=jax.ShapeDtypeStruct((K, AUG, B), jnp.float32),
    )(cbt)
    out = pl.pallas_call(
        _pq_kernel,
        grid=(v // BV,),
        in_specs=[
            pl.BlockSpec((BV, K * SUBD), lambda i: (i, 0)),
            pl.BlockSpec((K, AUG, B), lambda i: (0, 0, 0)),
        ],
        out_specs=pl.BlockSpec((BV, K), lambda i: (i, 0)),
        out_shape=jax.ShapeDtypeStruct((v, K), jnp.uint8),
        compiler_params=pltpu.CompilerParams(
            dimension_semantics=("parallel",),
        ),
    )(vectors, cbs)
    return out
